# baseline (speedup 1.0000x reference)
import functools

import jax
import jax.numpy as jnp
from jax import lax
from jax.experimental import pallas as pl
from jax.experimental.pallas import tpu as pltpu
from jax.experimental.pallas import tpu_sc as plsc

_CANDIDATE_TOPK = 8
_CENTER_RADIUS = 0.75
_DUP_RADIUS = 1.25
_VFL_ALPHA = 0.75
_VFL_GAMMA = 2.0
_BOX_WEIGHT = 2.5
_BN = 2000
_NEG_INF = float("-inf")
_BIG_I = 1 << 30


def _sig(x):
    return 1.0 / (1.0 + jnp.exp(-x))


def _gt_geom(gt_ref, ax, ay):
    x1 = gt_ref[0:1, :]
    y1 = gt_ref[1:2, :]
    x2 = gt_ref[2:3, :]
    y2 = gt_ref[3:4, :]
    inside = (ax >= x1) & (ax <= x2) & (ay >= y1) & (ay <= y2)
    cx = (x1 + x2) * 0.5
    cy = (y1 + y2) * 0.5
    hx = jnp.maximum((x2 - x1) * 0.5, 1.0)
    hy = jnp.maximum((y2 - y1) * 0.5, 1.0)
    dx = jnp.abs(ax - cx) / hx
    dy = jnp.abs(ay - cy) / hy
    dmax = jnp.maximum(dx, dy)
    d2 = dx * dx + dy * dy
    return inside, dmax, d2, (x1, y1, x2, y2)


def _anyc_body(ap_ref, gt_ref, out_ref, acc_ref):
    i = pl.program_id(0)
    ax = ap_ref[:, 0:1]
    ay = ap_ref[:, 1:2]
    inside, dmax, _, _ = _gt_geom(gt_ref, ax, ay)
    cc = inside & (dmax <= _CENTER_RADIUS)
    part = jnp.max(jnp.where(cc, 1.0, 0.0), axis=0, keepdims=True)

    @pl.when(i == 0)
    def _init():
        acc_ref[...] = jnp.zeros_like(acc_ref)

    acc_ref[...] = jnp.maximum(acc_ref[...], part)

    @pl.when(i == pl.num_programs(0) - 1)
    def _fin():
        out_ref[...] = acc_ref[...]


def _topk_body(scores_ref, boxes_ref, obj_ref, ap_ref, gt_ref, lab_ref,
               vcm_ref, anyc_ref, vals_out, idx_out, iou_out,
               sv_ref, si_ref, so_ref, *, nc, num_classes):
    i = pl.program_id(0)
    bn = scores_ref.shape[0]
    g = lab_ref.shape[1]

    ax = ap_ref[:, 0:1]
    ay = ap_ref[:, 1:2]
    inside, dmax, d2, (x1, y1, x2, y2) = _gt_geom(gt_ref, ax, ay)
    ccf = jnp.where(inside & (dmax <= _CENTER_RADIUS), 1.0, 0.0)
    insidef = jnp.where(inside, 1.0, 0.0)
    anycf = anyc_ref[...]
    candf = anycf * ccf + (1.0 - anycf) * insidef
    prior = jnp.exp(-0.5 * d2)

    px1 = boxes_ref[:, 0:1]
    py1 = boxes_ref[:, 1:2]
    px2 = boxes_ref[:, 2:3]
    py2 = boxes_ref[:, 3:4]
    ix1 = jnp.maximum(px1, x1)
    iy1 = jnp.maximum(py1, y1)
    ix2 = jnp.minimum(px2, x2)
    iy2 = jnp.minimum(py2, y2)
    inter = jnp.maximum(ix2 - ix1, 0.0) * jnp.maximum(iy2 - iy1, 0.0)
    area_a = jnp.maximum(px2 - px1, 0.0) * jnp.maximum(py2 - py1, 0.0)
    area_b = jnp.maximum(x2 - x1, 0.0) * jnp.maximum(y2 - y1, 0.0)
    iou = inter / (area_a + area_b - inter + 1e-7)

    lab = lab_ref[...]
    onehot = jnp.where(
        lax.broadcasted_iota(jnp.int32, (num_classes, g), 0) == lab, 1.0, 0.0)
    logit = jnp.dot(scores_ref[...], onehot, preferred_element_type=jnp.float32)
    cls_s = _sig(logit)
    po = _sig(obj_ref[...])
    vcm_g = jnp.dot(vcm_ref[...], onehot, preferred_element_type=jnp.float32)
    vgtf = jnp.where((lab >= 0) & (lab < num_classes), 1.0, 0.0) * jnp.where(
        vcm_g > 0.0, 1.0, 0.0)

    quality = jnp.sqrt(jnp.maximum(po * cls_s, 0.0))
    metric = quality * (iou * iou) * (prior * prior)
    m = jnp.where(candf * vgtf > 0.0, metric, _NEG_INF)
    gidx = i * bn + lax.broadcasted_iota(jnp.int32, (bn, g), 0)

    for k in range(_CANDIDATE_TOPK):
        mx = jnp.max(m, axis=0, keepdims=True)
        pick = m == mx
        cidx = jnp.min(jnp.where(pick, gidx, _BIG_I), axis=0, keepdims=True)
        sel = pick & (gidx == cidx)
        sv_ref[8 + k:9 + k, :] = mx
        si_ref[8 + k:9 + k, :] = cidx
        so_ref[8 + k:9 + k, :] = jnp.sum(
            jnp.where(sel, iou, 0.0), axis=0, keepdims=True)
        m = jnp.where(sel, _NEG_INF, m)

    @pl.when(i == 0)
    def _init():
        sv_ref[0:8, :] = jnp.full((8, g), _NEG_INF, jnp.float32)
        si_ref[0:8, :] = jnp.full((8, g), _BIG_I, jnp.int32)
        so_ref[0:8, :] = jnp.zeros((8, g), jnp.float32)

    v = sv_ref[...]
    ix = si_ref[...]
    io = so_ref[...]
    outv, outi, outo = [], [], []
    for k in range(_CANDIDATE_TOPK):
        mx = jnp.max(v, axis=0, keepdims=True)
        pick = v == mx
        cidx = jnp.min(jnp.where(pick, ix, _BIG_I), axis=0, keepdims=True)
        sel = pick & (ix == cidx)
        outv.append(mx)
        outi.append(cidx)
        outo.append(jnp.sum(jnp.where(sel, io, 0.0), axis=0, keepdims=True))
        v = jnp.where(sel, _NEG_INF, v)
    tv = jnp.concatenate(outv, axis=0)
    ti = jnp.concatenate(outi, axis=0)
    to = jnp.concatenate(outo, axis=0)
    sv_ref[0:8, :] = tv
    si_ref[0:8, :] = ti
    so_ref[0:8, :] = to

    @pl.when(i == nc - 1)
    def _fin():
        vals_out[...] = tv
        idx_out[...] = ti
        iou_out[...] = to


_NCHUNK = 50
_GPAD = 112


def _greedy_sc_body(vals_hbm, idx_hbm, iou_hbm, lab_hbm,
                    mp_hbm, mlab_hbm, movl_hbm, mval_hbm, gm_hbm, scal_hbm,
                    v_v, idx_v, iou_v, lab_v,
                    mp_v, mlab_v, movl_v, mval_v, gm_v, sc_v):
    cid = lax.axis_index("c")
    sid = lax.axis_index("s")

    @pl.when((cid == 0) & (sid == 0))
    def _work():
        pltpu.sync_copy(vals_hbm, v_v)
        pltpu.sync_copy(idx_hbm, idx_v)
        pltpu.sync_copy(iou_hbm, iou_v)
        pltpu.sync_copy(lab_hbm, lab_v)

        zf = jnp.zeros((16,), jnp.float32)
        zi = jnp.zeros((16,), jnp.int32)
        for c in range(8):
            mp_v[pl.ds(c * 16, 16)] = zi
            mlab_v[pl.ds(c * 16, 16)] = zi
            movl_v[pl.ds(c * 16, 16)] = zf
            mval_v[pl.ds(c * 16, 16)] = zf
        for c in range(_GPAD // 16):
            gm_v[pl.ds(c * 16, 16)] = zf

        iota16 = lax.iota(jnp.int32, 16)

        def _bf_max(v):
            for s in (8, 4, 2, 1):
                v = jnp.maximum(
                    v, v.at[iota16 ^ s].get(mode="promise_in_bounds"))
            return v

        def _bf_min_i(v):
            for s in (8, 4, 2, 1):
                v = jnp.minimum(
                    v, v.at[iota16 ^ s].get(mode="promise_in_bounds"))
            return v

        def sweep(takev, g0s, p0v):
            def sw(j, c):
                bmax, bflat = c
                dsl = pl.ds(j * 16, 16)
                vj = v_v[dsl]
                ij = idx_v[dsl]
                fj = j * 16 + iota16
                kill = takev & (((fj >> 3) == g0s) | (ij == p0v))
                vj2 = jnp.where(kill, _NEG_INF, vj)
                v_v[dsl] = vj2
                better = vj2 > bmax
                return (jnp.where(better, vj2, bmax),
                        jnp.where(better, fj, bflat))

            bmax, bflat = lax.fori_loop(
                0, _NCHUNK, sw,
                (jnp.full((16,), _NEG_INF, jnp.float32),
                 jnp.zeros((16,), jnp.int32)))
            m = _bf_max(bmax)
            fm = _bf_min_i(jnp.where(bmax == m, bflat, _BIG_I))
            return m, fm

        def body(t, st):
            mxv, fselv, bsumv, cntv = st
            takev = mxv > _NEG_INF
            fs = jnp.where(takev, fselv, 0)[0]
            g0s = fs >> 3
            c0 = (fs >> 4) << 4
            l0i = iota16 * 0 + (fs & 15)
            ich = idx_v[pl.ds(c0, 16)]
            och = iou_v[pl.ds(c0, 16)]
            p0v = ich.at[l0i].get(mode="promise_in_bounds")
            i0v = och.at[l0i].get(mode="promise_in_bounds")
            gch = lab_v[pl.ds((g0s >> 4) << 4, 16)]
            labv = gch.at[iota16 * 0 + (g0s & 15)].get(
                mode="promise_in_bounds")

            sel_t = (iota16 == (t & 15)) & takev
            dst = pl.ds((t >> 4) << 4, 16)
            mp_v[dst] = jnp.where(sel_t, p0v, mp_v[dst])
            mlab_v[dst] = jnp.where(sel_t, labv, mlab_v[dst])
            movl_v[dst] = jnp.where(sel_t, i0v, movl_v[dst])
            mval_v[dst] = jnp.where(sel_t, 1.0, mval_v[dst])
            sel_g = (iota16 == (g0s & 15)) & takev
            dsg = pl.ds((g0s >> 4) << 4, 16)
            gm_v[dsg] = jnp.where(sel_g, 1.0, gm_v[dsg])

            bsumv = bsumv + jnp.where(takev, 1.0 - i0v, 0.0)
            cntv = cntv + jnp.where(takev, 1.0, 0.0)
            nmx, nfs = sweep(takev, g0s, p0v)
            return (nmx, nfs, bsumv, cntv)

        mx0, fs0 = sweep(iota16 < 0, jnp.int32(0), zi)
        _, _, bsumv, cntv = lax.fori_loop(
            0, 100, body, (mx0, fs0, zf, zf))

        sc_v[...] = jnp.where(iota16 == 0, bsumv,
                              jnp.where(iota16 == 1, cntv, 0.0))
        pltpu.sync_copy(mp_v, mp_hbm)
        pltpu.sync_copy(mlab_v, mlab_hbm)
        pltpu.sync_copy(movl_v, movl_hbm)
        pltpu.sync_copy(mval_v, mval_hbm)
        pltpu.sync_copy(gm_v, gm_hbm)
        pltpu.sync_copy(sc_v, scal_hbm)


def _loss_body(scores_ref, obj_ref, ap_ref, gt_ref, mp_ref, mlab_ref,
               movl_ref, mval_ref, gmask_ref, out_ref, acc_ref, *, nc):
    i = pl.program_id(0)
    bn = scores_ref.shape[0]
    num_classes = scores_ref.shape[1]

    niota = mp_ref[...] * 0 + i * bn + lax.broadcasted_iota(
        jnp.int32, (bn, 128), 0)
    eq = (niota == mp_ref[...]) & (mval_ref[...] > 0.0)
    eqf = jnp.where(eq, 1.0, 0.0)
    fgf = jnp.max(eqf, axis=1, keepdims=True)
    mlabel = jnp.sum(jnp.where(eq, mlab_ref[...], 0), axis=1, keepdims=True)
    movl = jnp.sum(eqf * movl_ref[...], axis=1, keepdims=True)
    tval = jnp.maximum(movl, 0.1) * fgf

    x = scores_ref[...]
    eqc = lax.broadcasted_iota(jnp.int32, (bn, num_classes), 1) == mlabel
    t = jnp.where(eqc, tval, 0.0) * fgf
    prob = jax.nn.sigmoid(x)
    w = _VFL_ALPHA * prob * prob * (1.0 - t) + t
    bce = jnp.maximum(x, 0.0) - x * t + jnp.log1p(jnp.exp(-jnp.abs(x)))
    cls_sum = jnp.sum(bce * w)

    ax = ap_ref[:, 0:1]
    ay = ap_ref[:, 1:2]
    inside, dmax, _, _ = _gt_geom(gt_ref, ax, ay)
    dc = inside & (dmax <= _DUP_RADIUS) & (gmask_ref[...] > 0.0)
    dupany = jnp.max(jnp.where(dc, 1.0, 0.0), axis=1, keepdims=True)
    dup = dupany * (1.0 - fgf)

    ox = obj_ref[...]
    bobj = (jnp.maximum(ox, 0.0) - ox * fgf
            + jnp.log1p(jnp.exp(-jnp.abs(ox))))
    obj_sum = jnp.sum(bobj * (1.0 - dup))

    iota128 = lax.broadcasted_iota(jnp.int32, (1, 128), 1)
    part = (cls_sum * jnp.where(iota128 == 0, 1.0, 0.0)
            + obj_sum * jnp.where(iota128 == 1, 1.0, 0.0))

    @pl.when(i == 0)
    def _init():
        acc_ref[...] = jnp.zeros_like(acc_ref)

    acc_ref[...] = acc_ref[...] + part

    @pl.when(i == nc - 1)
    def _fin():
        out_ref[...] = acc_ref[...]


def kernel(pred_scores, pred_boxes, pred_objectness, anchor_points,
           gt_boxes, gt_labels, valid_class_mask):
    n, num_classes = pred_scores.shape
    g = gt_boxes.shape[0]
    nc = n // _BN
    obj2d = pred_objectness.reshape(n, 1)
    gt_t = gt_boxes.T
    lab2d = gt_labels.reshape(1, g)
    vcm2d = valid_class_mask.astype(jnp.float32).reshape(1, num_classes)

    anyc = pl.pallas_call(
        _anyc_body,
        grid=(nc,),
        in_specs=[
            pl.BlockSpec((_BN, 2), lambda i: (i, 0)),
            pl.BlockSpec((4, g), lambda i: (0, 0)),
        ],
        out_specs=pl.BlockSpec((1, g), lambda i: (0, 0)),
        out_shape=jax.ShapeDtypeStruct((1, g), jnp.float32),
        scratch_shapes=[pltpu.VMEM((1, g), jnp.float32)],
    )(anchor_points, gt_t)

    vals8, idx8, iou8 = pl.pallas_call(
        functools.partial(_topk_body, nc=nc, num_classes=num_classes),
        grid=(nc,),
        in_specs=[
            pl.BlockSpec((_BN, num_classes), lambda i: (i, 0)),
            pl.BlockSpec((_BN, 4), lambda i: (i, 0)),
            pl.BlockSpec((_BN, 1), lambda i: (i, 0)),
            pl.BlockSpec((_BN, 2), lambda i: (i, 0)),
            pl.BlockSpec((4, g), lambda i: (0, 0)),
            pl.BlockSpec((1, g), lambda i: (0, 0)),
            pl.BlockSpec((1, num_classes), lambda i: (0, 0)),
            pl.BlockSpec((1, g), lambda i: (0, 0)),
        ],
        out_specs=[
            pl.BlockSpec((8, g), lambda i: (0, 0)),
            pl.BlockSpec((8, g), lambda i: (0, 0)),
            pl.BlockSpec((8, g), lambda i: (0, 0)),
        ],
        out_shape=[
            jax.ShapeDtypeStruct((8, g), jnp.float32),
            jax.ShapeDtypeStruct((8, g), jnp.int32),
            jax.ShapeDtypeStruct((8, g), jnp.float32),
        ],
        scratch_shapes=[
            pltpu.VMEM((16, g), jnp.float32),
            pltpu.VMEM((16, g), jnp.int32),
            pltpu.VMEM((16, g), jnp.float32),
        ],
    )(pred_scores, pred_boxes, obj2d, anchor_points, gt_t, lab2d, vcm2d, anyc)

    valsf = vals8.T.reshape(8 * g)
    idxf = idx8.T.reshape(8 * g)
    iouf = iou8.T.reshape(8 * g)
    labf = jnp.pad(gt_labels.reshape(g).astype(jnp.int32), (0, _GPAD - g))

    greedy = functools.partial(
        pl.kernel,
        mesh=plsc.VectorSubcoreMesh(core_axis_name="c", subcore_axis_name="s"),
        out_type=[
            jax.ShapeDtypeStruct((128,), jnp.int32),
            jax.ShapeDtypeStruct((128,), jnp.int32),
            jax.ShapeDtypeStruct((128,), jnp.float32),
            jax.ShapeDtypeStruct((128,), jnp.float32),
            jax.ShapeDtypeStruct((_GPAD,), jnp.float32),
            jax.ShapeDtypeStruct((16,), jnp.float32),
        ],
        scratch_types=[
            pltpu.VMEM((8 * g,), jnp.float32),
            pltpu.VMEM((8 * g,), jnp.int32),
            pltpu.VMEM((8 * g,), jnp.float32),
            pltpu.VMEM((_GPAD,), jnp.int32),
            pltpu.VMEM((128,), jnp.int32),
            pltpu.VMEM((128,), jnp.int32),
            pltpu.VMEM((128,), jnp.float32),
            pltpu.VMEM((128,), jnp.float32),
            pltpu.VMEM((_GPAD,), jnp.float32),
            pltpu.VMEM((16,), jnp.float32),
        ],
    )(_greedy_sc_body)
    mp1, mlab1, movl1, mval1, gm1, scal1 = greedy(valsf, idxf, iouf, labf)
    mp = mp1.reshape(1, 128)
    mlab = mlab1.reshape(1, 128)
    movl = movl1.reshape(1, 128)
    mval = mval1.reshape(1, 128)
    gmask = gm1[:g].reshape(1, g)
    scal = scal1.reshape(1, 16)

    sums = pl.pallas_call(
        functools.partial(_loss_body, nc=nc),
        grid=(nc,),
        in_specs=[
            pl.BlockSpec((_BN, num_classes), lambda i: (i, 0)),
            pl.BlockSpec((_BN, 1), lambda i: (i, 0)),
            pl.BlockSpec((_BN, 2), lambda i: (i, 0)),
            pl.BlockSpec((4, g), lambda i: (0, 0)),
            pl.BlockSpec((1, 128), lambda i: (0, 0)),
            pl.BlockSpec((1, 128), lambda i: (0, 0)),
            pl.BlockSpec((1, 128), lambda i: (0, 0)),
            pl.BlockSpec((1, 128), lambda i: (0, 0)),
            pl.BlockSpec((1, g), lambda i: (0, 0)),
        ],
        out_specs=pl.BlockSpec((1, 128), lambda i: (0, 0)),
        out_shape=jax.ShapeDtypeStruct((1, 128), jnp.float32),
        scratch_shapes=[pltpu.VMEM((1, 128), jnp.float32)],
    )(pred_scores, obj2d, anchor_points, gt_t, mp, mlab, movl, mval, gmask)

    box_sum = scal[0, 0]
    cnt = scal[0, 1]
    num_fg = jnp.maximum(cnt, 1.0)
    return (sums[0, 0] + _BOX_WEIGHT * box_sum + sums[0, 1]) / num_fg
